# Initial kernel scaffold; baseline (speedup 1.0000x reference)
import jax
import jax.numpy as jnp
from jax.experimental import pallas as pl


def kernel(tile_feats, piece_feats, tile_edge_index, piece_to_tile, tile_to_piece, global_feats, params):
    raise NotImplementedError("write your pallas kernel here")


# speedup vs baseline: 2.8489x; 2.8489x over previous
import functools

import jax
import jax.numpy as jnp
from jax import lax
from jax.experimental import pallas as pl
from jax.experimental.pallas import tpu as pltpu
from jax.experimental.pallas import tpu_sc as plsc

H = 64
HH = 32
EK = 128
NC = 2
NS = 16


def _ceil_to(x, m):
    return (x + m - 1) // m * m


def _dense_body(x_ref, w_ref, b_ref, o_ref, *, relu):
    acc = jnp.dot(x_ref[...], w_ref[...], preferred_element_type=jnp.float32)
    acc = acc + b_ref[...]
    o_ref[...] = jnp.maximum(acc, 0.0) if relu else acc


def _tc_dense(x, w, b, n_rows=None, relu=True, block=1000):
    n = x.shape[0] if n_rows is None else n_rows
    k = x.shape[1]
    grid = n // block
    return pl.pallas_call(
        functools.partial(_dense_body, relu=relu),
        grid=(grid,),
        in_specs=[
            pl.BlockSpec((block, k), lambda i: (i, 0)),
            pl.BlockSpec((k, H), lambda i: (0, 0)),
            pl.BlockSpec((1, H), lambda i: (0, 0)),
        ],
        out_specs=pl.BlockSpec((block, H), lambda i: (i, 0)),
        out_shape=jax.ShapeDtypeStruct((n, H), jnp.float32),
    )(x, w, b.reshape(1, H))


def _matmul_nobias(x, w, n_rows=None, block=1000):
    n = x.shape[0] if n_rows is None else n_rows
    k = x.shape[1]

    def body(x_ref, w_ref, o_ref):
        o_ref[...] = jnp.dot(x_ref[...], w_ref[...],
                             preferred_element_type=jnp.float32)

    return pl.pallas_call(
        body,
        grid=(n // block,),
        in_specs=[
            pl.BlockSpec((block, k), lambda i: (i, 0)),
            pl.BlockSpec((k, H), lambda i: (0, 0)),
        ],
        out_specs=pl.BlockSpec((block, H), lambda i: (i, 0)),
        out_shape=jax.ShapeDtypeStruct((n, H), jnp.float32),
    )(x, w)


def _upd_body(h_ref, a0_ref, a1_ref, c0_ref, c1_ref, bm_ref, wu_ref, bu_ref,
              o_ref):
    cnt = c0_ref[0, :, :1] + c1_ref[0, :, :1]
    inv = 1.0 / jnp.maximum(cnt, 1.0)
    m = (cnt > 0.0).astype(jnp.float32)
    bm = bm_ref[...]
    r0 = a0_ref[0] * inv + m * bm[:, :HH]
    r1 = a1_ref[0] * inv + m * bm[:, HH:]
    wu = wu_ref[...]
    acc = jnp.dot(h_ref[...], wu[:H], preferred_element_type=jnp.float32)
    acc += jnp.dot(r0, wu[H:H + HH], preferred_element_type=jnp.float32)
    acc += jnp.dot(r1, wu[H + HH:], preferred_element_type=jnp.float32)
    o_ref[...] = jnp.maximum(acc + bu_ref[...], 0.0)


def _tc_update(h, agg, cnt2, b_msg, w_upd, b_upd, block=1000):
    n = h.shape[0]
    return pl.pallas_call(
        _upd_body,
        grid=(n // block,),
        in_specs=[
            pl.BlockSpec((block, H), lambda i: (i, 0)),
            pl.BlockSpec((1, block, HH), lambda i: (0, i, 0)),
            pl.BlockSpec((1, block, HH), lambda i: (1, i, 0)),
            pl.BlockSpec((1, block, 16), lambda i: (0, i, 0)),
            pl.BlockSpec((1, block, 16), lambda i: (1, i, 0)),
            pl.BlockSpec((1, H), lambda i: (0, 0)),
            pl.BlockSpec((2 * H, H), lambda i: (0, 0)),
            pl.BlockSpec((1, H), lambda i: (0, 0)),
        ],
        out_specs=pl.BlockSpec((block, H), lambda i: (i, 0)),
        out_shape=jax.ShapeDtypeStruct((n, H), jnp.float32),
    )(h, agg, agg, cnt2, cnt2, b_msg.reshape(1, H), w_upd,
      b_upd.reshape(1, H))


def _colsum_body(x_ref, o_ref):
    @pl.when(pl.program_id(0) == 0)
    def _init():
        o_ref[...] = jnp.zeros_like(o_ref)

    o_ref[...] += jnp.sum(x_ref[...], axis=0, keepdims=True)


def _tc_colsum(x, block=1000):
    n = x.shape[0]
    return pl.pallas_call(
        _colsum_body,
        grid=(n // block,),
        in_specs=[pl.BlockSpec((block, H), lambda i: (i, 0))],
        out_specs=pl.BlockSpec((1, H), lambda i: (0, 0)),
        out_shape=jax.ShapeDtypeStruct((1, H), jnp.float32),
    )(x)


def _mlp_body(st_ref, sp_ref, g_ref, w1_ref, b1_ref, w2_ref, b2_ref, w3_ref,
              b3_ref, o_ref, *, nt, np_):
    t = st_ref[...] * (1.0 / nt)
    p = sp_ref[...] * (1.0 / np_)
    comb = jnp.concatenate([t, p, g_ref[...]], axis=1)
    h = jnp.dot(comb, w1_ref[...], preferred_element_type=jnp.float32)
    h = jnp.maximum(h + b1_ref[...], 0.0)
    h = jnp.dot(h, w2_ref[...], preferred_element_type=jnp.float32)
    h = jnp.maximum(h + b2_ref[...], 0.0)
    h = jnp.dot(h, w3_ref[...], preferred_element_type=jnp.float32)
    o_ref[...] = h + b3_ref[...]


def _tc_readout(sum_t, sum_p, gfeat, ro, nt, np_):
    (w1, b1), (w2, b2), (w3, b3) = ro
    full = lambda s: pl.BlockSpec(s, lambda: tuple(0 for _ in s))
    return pl.pallas_call(
        functools.partial(_mlp_body, nt=float(nt), np_=float(np_)),
        in_specs=[full((1, H)), full((1, H)), full((1, 4)),
                  full(w1.shape), full((1, H)),
                  full(w2.shape), full((1, 32)),
                  full(w3.shape), full((1, 1))],
        out_specs=full((1, 1)),
        out_shape=jax.ShapeDtypeStruct((1, 1), jnp.float32),
    )(sum_t, sum_p, gfeat.reshape(1, 4), w1, b1.reshape(1, H),
      w2, b2.reshape(1, 32), w3, b3.reshape(1, 1))


@functools.cache
def _mesh():
    return plsc.VectorSubcoreMesh(core_axis_name="c", subcore_axis_name="s")


def _sc_segsum(table2, src2, dst, n_acc):
    e_pad = dst.shape[0]
    epw = e_pad // NS
    rpw = n_acc // NS
    zeros = jnp.zeros((rpw, HH), jnp.float32)

    @functools.partial(
        pl.kernel,
        out_type=jax.ShapeDtypeStruct((NC, n_acc, HH), jnp.float32),
        mesh=_mesh(),
        scratch_types=[
            pltpu.VMEM((EK,), jnp.int32),
            pltpu.VMEM((EK,), jnp.int32),
            pltpu.VMEM((EK, HH), jnp.float32),
            pltpu.VMEM_SHARED((n_acc, HH), jnp.float32),
            pltpu.SemaphoreType.DMA,
        ],
        compiler_params=pltpu.CompilerParams(use_tc_tiling_on_sc=False),
    )
    def k(table_hbm, src_hbm, dst_hbm, z_hbm, out_hbm,
          src_v, dst_v, rows_v, acc_sh, sem):
        cid = lax.axis_index("c")
        sid = lax.axis_index("s")
        pltpu.sync_copy(z_hbm, acc_sh.at[pl.ds(sid * rpw, rpw), :])
        plsc.subcore_barrier()

        base = sid * epw

        def step(t, _):
            off = base + t * EK
            pltpu.sync_copy(src_hbm.at[cid, pl.ds(off, EK)], src_v)
            pltpu.sync_copy(dst_hbm.at[pl.ds(off, EK)], dst_v)
            pltpu.async_copy(table_hbm.at[src_v], rows_v, sem).wait()
            pltpu.sync_copy(rows_v, acc_sh.at[dst_v], add=True)
            return 0

        lax.fori_loop(0, epw // EK, step, 0, unroll=False)
        plsc.subcore_barrier()
        pltpu.sync_copy(acc_sh.at[pl.ds(sid * rpw, rpw), :],
                        out_hbm.at[cid, pl.ds(sid * rpw, rpw), :])

    return k(table2, src2, dst, zeros)


def _sc_counts(dst, n_acc):
    e_pad = dst.shape[0]
    epw = e_pad // (NC * NS)
    rpw = n_acc // NS
    zeros = jnp.zeros((rpw, 16), jnp.float32)
    ones = jnp.ones((EK, 16), jnp.float32)

    @functools.partial(
        pl.kernel,
        out_type=jax.ShapeDtypeStruct((NC, n_acc, 16), jnp.float32),
        mesh=_mesh(),
        scratch_types=[
            pltpu.VMEM((EK,), jnp.int32),
            pltpu.VMEM((EK, 16), jnp.float32),
            pltpu.VMEM_SHARED((n_acc, 16), jnp.float32),
        ],
        compiler_params=pltpu.CompilerParams(use_tc_tiling_on_sc=False),
    )
    def k(dst_hbm, z_hbm, ones_hbm, out_hbm, dst_v, ones_v, acc_sh):
        cid = lax.axis_index("c")
        sid = lax.axis_index("s")
        pltpu.sync_copy(z_hbm, acc_sh.at[pl.ds(sid * rpw, rpw), :])
        pltpu.sync_copy(ones_hbm, ones_v)
        plsc.subcore_barrier()

        base = (cid * NS + sid) * epw

        def step(t, _):
            off = base + t * EK
            pltpu.sync_copy(dst_hbm.at[pl.ds(off, EK)], dst_v)
            pltpu.sync_copy(ones_v, acc_sh.at[dst_v], add=True)
            return 0

        lax.fori_loop(0, epw // EK, step, 0, unroll=False)
        plsc.subcore_barrier()
        pltpu.sync_copy(acc_sh.at[pl.ds(sid * rpw, rpw), :],
                        out_hbm.at[cid, pl.ds(sid * rpw, rpw), :])

    return k(dst, zeros, ones)


def _prep_edges(edge_index, n_dst):
    e = edge_index.shape[1]
    e_pad = _ceil_to(e, NC * NS * EK)
    pad = e_pad - e
    src = jnp.concatenate([edge_index[0], jnp.zeros((pad,), jnp.int32)])
    dst = jnp.concatenate([edge_index[1],
                           jnp.full((pad,), n_dst, jnp.int32)])
    src2 = jnp.stack([2 * src, 2 * src + 1])
    return src2, dst


def _mean_msgs(h_src, w_msg, src2, dst, n_acc, n_src=None):
    tmp = _matmul_nobias(h_src, w_msg, n_rows=n_src)
    table2 = tmp.reshape(-1, HH)
    return _sc_segsum(table2, src2, dst, n_acc)


def kernel(tile_feats, piece_feats, tile_edge_index, piece_to_tile,
           tile_to_piece, global_feats, params):
    num_tiles = tile_feats.shape[0]
    num_pieces = piece_feats.shape[0]
    n_acc_t = _ceil_to(num_tiles + 1, NS * 8)
    n_acc_p = _ceil_to(num_pieces + 1, NS * 8)

    t2p_src2, t2p_dst = _prep_edges(tile_to_piece, num_pieces)
    p2t_src2, p2t_dst = _prep_edges(piece_to_tile, num_tiles)
    t2t_src2, t2t_dst = _prep_edges(tile_edge_index, num_tiles)

    cnt_t2p = _sc_counts(t2p_dst, n_acc_p)
    cnt_p2t = _sc_counts(p2t_dst, n_acc_t)
    cnt_t2t = _sc_counts(t2t_dst, n_acc_t)

    tile_h = _tc_dense(tile_feats, *params['tile_embed'])
    piece_h = _tc_dense(piece_feats, *params['piece_embed'])

    for lp in params['mp']:
        agg = _mean_msgs(tile_h, lp['t2p'][0], t2p_src2, t2p_dst, n_acc_p,
                         n_src=num_pieces)
        piece_h = _tc_update(piece_h, agg, cnt_t2p, lp['t2p'][1],
                             lp['p_upd'][0], lp['p_upd'][1])
        agg = _mean_msgs(piece_h, lp['p2t'][0], p2t_src2, p2t_dst, n_acc_t)
        tile_h = _tc_update(tile_h, agg, cnt_p2t, lp['p2t'][1],
                            lp['t_upd_p'][0], lp['t_upd_p'][1])
        agg = _mean_msgs(tile_h, lp['t2t'][0], t2t_src2, t2t_dst, n_acc_t)
        tile_h = _tc_update(tile_h, agg, cnt_t2t, lp['t2t'][1],
                            lp['t_upd_t'][0], lp['t_upd_t'][1])

    sum_t = _tc_colsum(tile_h)
    sum_p = _tc_colsum(piece_h)
    out = _tc_readout(sum_t, sum_p, global_feats, params['readout'],
                      num_tiles, num_pieces)
    return out.reshape(())


# speedup vs baseline: 3.5293x; 1.2389x over previous
import functools

import jax
import jax.numpy as jnp
from jax import lax
from jax.experimental import pallas as pl
from jax.experimental.pallas import tpu as pltpu
from jax.experimental.pallas import tpu_sc as plsc

H = 64
HH = 32
EK = 128
NC = 2
NS = 16


def _ceil_to(x, m):
    return (x + m - 1) // m * m


def _dense_body(x_ref, w_ref, b_ref, o_ref, *, relu):
    acc = jnp.dot(x_ref[...], w_ref[...], preferred_element_type=jnp.float32)
    acc = acc + b_ref[...]
    o_ref[...] = jnp.maximum(acc, 0.0) if relu else acc


def _tc_dense(x, w, b, n_rows=None, relu=True, block=1000):
    n = x.shape[0] if n_rows is None else n_rows
    k = x.shape[1]
    grid = n // block
    return pl.pallas_call(
        functools.partial(_dense_body, relu=relu),
        grid=(grid,),
        in_specs=[
            pl.BlockSpec((block, k), lambda i: (i, 0)),
            pl.BlockSpec((k, H), lambda i: (0, 0)),
            pl.BlockSpec((1, H), lambda i: (0, 0)),
        ],
        out_specs=pl.BlockSpec((block, H), lambda i: (i, 0)),
        out_shape=jax.ShapeDtypeStruct((n, H), jnp.float32),
    )(x, w, b.reshape(1, H))


def _matmul_nobias(x, w, n_rows=None, block=1000):
    n = x.shape[0] if n_rows is None else n_rows
    k = x.shape[1]

    def body(x_ref, w_ref, o_ref):
        o_ref[...] = jnp.dot(x_ref[...], w_ref[...],
                             preferred_element_type=jnp.float32)

    return pl.pallas_call(
        body,
        grid=(n // block,),
        in_specs=[
            pl.BlockSpec((block, k), lambda i: (i, 0)),
            pl.BlockSpec((k, H), lambda i: (0, 0)),
        ],
        out_specs=pl.BlockSpec((block, H), lambda i: (i, 0)),
        out_shape=jax.ShapeDtypeStruct((n, H), jnp.float32),
    )(x, w)


def _upd_body(h_ref, a0_ref, a1_ref, c0_ref, c1_ref, bm_ref, wu_ref, bu_ref,
              o_ref):
    cnt = c0_ref[0, :, :1] + c1_ref[0, :, :1]
    inv = 1.0 / jnp.maximum(cnt, 1.0)
    m = (cnt > 0.0).astype(jnp.float32)
    bm = bm_ref[...]
    r0 = (a0_ref[0] * inv + bm[:, :HH]) * m
    r1 = (a1_ref[0] * inv + bm[:, HH:]) * m
    wu = wu_ref[...]
    acc = jnp.dot(h_ref[...], wu[:H], preferred_element_type=jnp.float32)
    acc += jnp.dot(r0, wu[H:H + HH], preferred_element_type=jnp.float32)
    acc += jnp.dot(r1, wu[H + HH:], preferred_element_type=jnp.float32)
    o_ref[...] = jnp.maximum(acc + bu_ref[...], 0.0)


def _tc_update(h, agg, cnt2, b_msg, w_upd, b_upd, block=1000):
    n = h.shape[0]
    return pl.pallas_call(
        _upd_body,
        grid=(n // block,),
        in_specs=[
            pl.BlockSpec((block, H), lambda i: (i, 0)),
            pl.BlockSpec((1, block, HH), lambda i: (0, i, 0)),
            pl.BlockSpec((1, block, HH), lambda i: (1, i, 0)),
            pl.BlockSpec((1, block, 16), lambda i: (0, i, 0)),
            pl.BlockSpec((1, block, 16), lambda i: (1, i, 0)),
            pl.BlockSpec((1, H), lambda i: (0, 0)),
            pl.BlockSpec((2 * H, H), lambda i: (0, 0)),
            pl.BlockSpec((1, H), lambda i: (0, 0)),
        ],
        out_specs=pl.BlockSpec((block, H), lambda i: (i, 0)),
        out_shape=jax.ShapeDtypeStruct((n, H), jnp.float32),
    )(h, agg, agg, cnt2, cnt2, b_msg.reshape(1, H), w_upd,
      b_upd.reshape(1, H))


def _colsum_body(x_ref, o_ref):
    @pl.when(pl.program_id(0) == 0)
    def _init():
        o_ref[...] = jnp.zeros_like(o_ref)

    o_ref[...] += jnp.sum(x_ref[...], axis=0, keepdims=True)


def _tc_colsum(x, block=1000):
    n = x.shape[0]
    return pl.pallas_call(
        _colsum_body,
        grid=(n // block,),
        in_specs=[pl.BlockSpec((block, H), lambda i: (i, 0))],
        out_specs=pl.BlockSpec((1, H), lambda i: (0, 0)),
        out_shape=jax.ShapeDtypeStruct((1, H), jnp.float32),
    )(x)


def _mlp_body(st_ref, sp_ref, g_ref, w1_ref, b1_ref, w2_ref, b2_ref, w3_ref,
              b3_ref, o_ref, *, nt, np_):
    t = st_ref[...] * (1.0 / nt)
    p = sp_ref[...] * (1.0 / np_)
    comb = jnp.concatenate([t, p, g_ref[...]], axis=1)
    h = jnp.dot(comb, w1_ref[...], preferred_element_type=jnp.float32)
    h = jnp.maximum(h + b1_ref[...], 0.0)
    h = jnp.dot(h, w2_ref[...], preferred_element_type=jnp.float32)
    h = jnp.maximum(h + b2_ref[...], 0.0)
    h = jnp.dot(h, w3_ref[...], preferred_element_type=jnp.float32)
    o_ref[...] = h + b3_ref[...]


def _tc_readout(sum_t, sum_p, gfeat, ro, nt, np_):
    (w1, b1), (w2, b2), (w3, b3) = ro
    full = lambda s: pl.BlockSpec(s, lambda: tuple(0 for _ in s))
    return pl.pallas_call(
        functools.partial(_mlp_body, nt=float(nt), np_=float(np_)),
        in_specs=[full((1, H)), full((1, H)), full((1, 4)),
                  full(w1.shape), full((1, H)),
                  full(w2.shape), full((1, 32)),
                  full(w3.shape), full((1, 1))],
        out_specs=full((1, 1)),
        out_shape=jax.ShapeDtypeStruct((1, 1), jnp.float32),
    )(sum_t, sum_p, gfeat.reshape(1, 4), w1, b1.reshape(1, H),
      w2, b2.reshape(1, 32), w3, b3.reshape(1, 1))


@functools.cache
def _mesh():
    return plsc.VectorSubcoreMesh(core_axis_name="c", subcore_axis_name="s")


NCH_C = 4


def _sc_segsum(table2, src2, dst, n_acc, nch):
    e_rows = dst.shape[0]
    rows_per_sub = e_rows // NS
    nblocks = rows_per_sub // nch
    rpw = n_acc // NS
    zeros = jnp.zeros((rpw, HH), jnp.float32)

    @functools.partial(
        pl.kernel,
        out_type=jax.ShapeDtypeStruct((NC, n_acc, HH), jnp.float32),
        mesh=_mesh(),
        scratch_types=[
            pltpu.VMEM((nch, EK), jnp.int32),
            pltpu.VMEM((nch, EK), jnp.int32),
            pltpu.VMEM((nch, EK, HH), jnp.float32),
            pltpu.VMEM_SHARED((n_acc, HH), jnp.float32),
            pltpu.SemaphoreType.DMA,
            pltpu.SemaphoreType.DMA,
        ],
        compiler_params=pltpu.CompilerParams(use_tc_tiling_on_sc=False),
    )
    def k(table_hbm, src_hbm, dst_hbm, z_hbm, out_hbm,
          src_v, dst_v, rows_v, acc_sh, gsem, ssem):
        cid = lax.axis_index("c")
        sid = lax.axis_index("s")
        pltpu.sync_copy(z_hbm, acc_sh.at[pl.ds(sid * rpw, rpw), :])
        plsc.subcore_barrier()

        base = sid * rows_per_sub

        def block(b, _):
            row0 = base + b * nch
            pltpu.sync_copy(src_hbm.at[cid, pl.ds(row0, nch), :], src_v)
            pltpu.sync_copy(dst_hbm.at[pl.ds(row0, nch), :], dst_v)
            gd = [pltpu.async_copy(table_hbm.at[src_v.at[j]], rows_v.at[j],
                                   gsem) for j in range(nch)]
            for d in gd:
                d.wait()
            sd = [pltpu.async_copy(rows_v.at[j], acc_sh.at[dst_v.at[j]],
                                   ssem, add=True) for j in range(nch)]
            for d in sd:
                d.wait()
            return 0

        lax.fori_loop(0, nblocks, block, 0, unroll=False)
        plsc.subcore_barrier()
        pltpu.sync_copy(acc_sh.at[pl.ds(sid * rpw, rpw), :],
                        out_hbm.at[cid, pl.ds(sid * rpw, rpw), :])

    return k(table2, src2, dst, zeros)


def _sc_counts(dst, n_acc):
    e_rows = dst.shape[0]
    rows_per_w = e_rows // (NC * NS)
    nblocks = rows_per_w // NCH_C
    rpw = n_acc // NS
    zeros = jnp.zeros((rpw, 16), jnp.float32)
    ones = jnp.ones((EK, 16), jnp.float32)

    @functools.partial(
        pl.kernel,
        out_type=jax.ShapeDtypeStruct((NC, n_acc, 16), jnp.float32),
        mesh=_mesh(),
        scratch_types=[
            pltpu.VMEM((NCH_C, EK), jnp.int32),
            pltpu.VMEM((EK, 16), jnp.float32),
            pltpu.VMEM_SHARED((n_acc, 16), jnp.float32),
            pltpu.SemaphoreType.DMA,
        ],
        compiler_params=pltpu.CompilerParams(use_tc_tiling_on_sc=False),
    )
    def k(dst_hbm, z_hbm, ones_hbm, out_hbm, dst_v, ones_v, acc_sh, ssem):
        cid = lax.axis_index("c")
        sid = lax.axis_index("s")
        pltpu.sync_copy(z_hbm, acc_sh.at[pl.ds(sid * rpw, rpw), :])
        pltpu.sync_copy(ones_hbm, ones_v)
        plsc.subcore_barrier()

        base = (cid * NS + sid) * rows_per_w

        def block(b, _):
            row0 = base + b * NCH_C
            pltpu.sync_copy(dst_hbm.at[pl.ds(row0, NCH_C), :], dst_v)
            sd = [pltpu.async_copy(ones_v, acc_sh.at[dst_v.at[j]],
                                   ssem, add=True) for j in range(NCH_C)]
            for d in sd:
                d.wait()
            return 0

        lax.fori_loop(0, nblocks, block, 0, unroll=False)
        plsc.subcore_barrier()
        pltpu.sync_copy(acc_sh.at[pl.ds(sid * rpw, rpw), :],
                        out_hbm.at[cid, pl.ds(sid * rpw, rpw), :])

    return k(dst, zeros, ones)


def _prep_edges(edge_index, n_dst, nch):
    e = edge_index.shape[1]
    e_pad = _ceil_to(e, NS * EK * max(nch, 2 * NCH_C))
    pad = e_pad - e
    src = jnp.concatenate([edge_index[0], jnp.zeros((pad,), jnp.int32)])
    dst = jnp.concatenate([edge_index[1],
                           jnp.full((pad,), n_dst, jnp.int32)])
    src2 = jnp.stack([2 * src, 2 * src + 1])
    return src2.reshape(2, e_pad // EK, EK), dst.reshape(e_pad // EK, EK)


def _mean_msgs(h_src, w_msg, src2, dst, n_acc, nch, n_src=None):
    tmp = _matmul_nobias(h_src, w_msg, n_rows=n_src)
    table2 = tmp.reshape(-1, HH)
    return _sc_segsum(table2, src2, dst, n_acc, nch)


def kernel(tile_feats, piece_feats, tile_edge_index, piece_to_tile,
           tile_to_piece, global_feats, params):
    num_tiles = tile_feats.shape[0]
    num_pieces = piece_feats.shape[0]
    n_acc_t = _ceil_to(num_tiles + 1, NS * 8)
    n_acc_p = _ceil_to(num_pieces + 1, NS * 8)

    nch_t2p, nch_p2t, nch_t2t = 16, 4, 4

    t2p_src2, t2p_dst = _prep_edges(tile_to_piece, num_pieces, nch_t2p)
    p2t_src2, p2t_dst = _prep_edges(piece_to_tile, num_tiles, nch_p2t)
    t2t_src2, t2t_dst = _prep_edges(tile_edge_index, num_tiles, nch_t2t)

    cnt_t2p = _sc_counts(t2p_dst, n_acc_p)
    cnt_p2t = _sc_counts(p2t_dst, n_acc_t)
    cnt_t2t = _sc_counts(t2t_dst, n_acc_t)

    tile_h = _tc_dense(tile_feats, *params['tile_embed'])
    piece_h = _tc_dense(piece_feats, *params['piece_embed'])

    for lp in params['mp']:
        agg = _mean_msgs(tile_h, lp['t2p'][0], t2p_src2, t2p_dst, n_acc_p,
                         nch_t2p, n_src=num_pieces)
        piece_h = _tc_update(piece_h, agg, cnt_t2p, lp['t2p'][1],
                             lp['p_upd'][0], lp['p_upd'][1])
        agg = _mean_msgs(piece_h, lp['p2t'][0], p2t_src2, p2t_dst, n_acc_t,
                         nch_p2t)
        tile_h = _tc_update(tile_h, agg, cnt_p2t, lp['p2t'][1],
                            lp['t_upd_p'][0], lp['t_upd_p'][1])
        agg = _mean_msgs(tile_h, lp['t2t'][0], t2t_src2, t2t_dst, n_acc_t,
                         nch_t2t)
        tile_h = _tc_update(tile_h, agg, cnt_t2t, lp['t2t'][1],
                            lp['t_upd_t'][0], lp['t_upd_t'][1])

    sum_t = _tc_colsum(tile_h)
    sum_p = _tc_colsum(piece_h)
    out = _tc_readout(sum_t, sum_p, global_feats, params['readout'],
                      num_tiles, num_pieces)
    return out.reshape(())
